# baseline (speedup 1.0000x reference)
import functools

import jax
import jax.numpy as jnp
from jax import lax
from jax.experimental import pallas as pl
from jax.experimental.pallas import tpu as pltpu
from jax.experimental.pallas import tpu_sc as plsc

NC, NS, L = 2, 16, 16
NW = NC * NS
B = 16384
D = 64
BPW = B // NW
CW = 128
NCHUNK = BPW // CW
GPC = CW // L

_LOG1P_C = (
    0.999998763883492,
    -0.4998719252774506,
    0.33112058367396247,
    -0.2351488240931608,
    0.14943483645817232,
    -0.06658820573216659,
    0.01420285926015165,
)


def _softplus(x):
    z = jnp.exp(-jnp.abs(x))
    p = jnp.float32(_LOG1P_C[-1])
    for c in _LOG1P_C[-2::-1]:
        p = p * z + jnp.float32(c)
    return jnp.maximum(x, jnp.float32(0.0)) + p * z


def _irt_body(sid_hbm, qid_hbm, bgat_hbm, theta_hbm, alpha_hbm, out_hbm,
              sid_v, qid_v, beta_v, theta_v, alpha_v, out_v, sem0, sem1):
    wid = lax.axis_index("s") * NC + lax.axis_index("c")

    pltpu.sync_copy(sid_hbm.at[pl.ds(wid * NCHUNK, NCHUNK)], sid_v)
    pltpu.sync_copy(qid_hbm.at[pl.ds(wid * NCHUNK, NCHUNK)], qid_v)
    pltpu.sync_copy(bgat_hbm.at[pl.ds(wid * NCHUNK, NCHUNK)], beta_v)

    sems = (sem0, sem1)

    def fire(j):
        s = sems[j % 2]
        cp_t = pltpu.async_copy(
            theta_hbm.at[sid_v.at[j]], theta_v.at[pl.ds(j * CW, CW)], s)
        cp_a = pltpu.async_copy(
            alpha_hbm.at[qid_v.at[j]], alpha_v.at[pl.ds(j * CW, CW)], s)
        return cp_t, cp_a

    iota = lax.iota(jnp.int32, L)
    inflight = fire(0)
    for j in range(NCHUNK):
        nxt = fire(j + 1) if j + 1 < NCHUNK else None
        for c in inflight:
            c.wait()
        inflight = nxt

        def group(g, _):
            rows = iota + (j * CW + g * L)
            acc = jnp.zeros((L,), jnp.float32)
            for d in range(D):
                cols = jnp.full((L,), d, jnp.int32)
                th = plsc.load_gather(theta_v, [rows, cols])
                al = plsc.load_gather(alpha_v, [rows, cols])
                acc = acc + _softplus(al) * th
            bta = plsc.load_gather(
                beta_v, [lax.shift_right_logical(rows, 7),
                         lax.bitwise_and(rows, jnp.full((L,), CW - 1, jnp.int32))])
            logit = acc + bta
            pred = jnp.float32(1.0) / (jnp.float32(1.0) + jnp.exp(-logit))
            plsc.store_scatter(out_v, [rows], pred)
            return 0

        lax.fori_loop(0, GPC, group, 0)

    pltpu.sync_copy(out_v, out_hbm.at[pl.ds(wid * BPW, BPW)])


@jax.jit
def _irt(sid, qid, bgat, theta_w, alpha_w):
    mesh = plsc.VectorSubcoreMesh(
        core_axis_name="c", subcore_axis_name="s",
        num_cores=NC, num_subcores=NS)
    run = pl.kernel(
        _irt_body,
        out_type=jax.ShapeDtypeStruct((B,), jnp.float32),
        mesh=mesh,
        compiler_params=pltpu.CompilerParams(
            needs_layout_passes=False, use_tc_tiling_on_sc=False),
        scratch_types=[
            pltpu.VMEM((NCHUNK, CW), jnp.int32),
            pltpu.VMEM((NCHUNK, CW), jnp.int32),
            pltpu.VMEM((NCHUNK, CW), jnp.float32),
            pltpu.VMEM((BPW, D), jnp.float32),
            pltpu.VMEM((BPW, D), jnp.float32),
            pltpu.VMEM((BPW,), jnp.float32),
            pltpu.SemaphoreType.DMA,
            pltpu.SemaphoreType.DMA,
        ],
    )
    return run(sid, qid, bgat, theta_w, alpha_w)


def kernel(student_ids, question_ids, theta_w, alpha_w, beta_w):
    sid = student_ids.astype(jnp.int32).reshape(B // CW, CW)
    qid = question_ids.astype(jnp.int32).reshape(B // CW, CW)
    bgat = jnp.take(beta_w, question_ids, axis=0).reshape(B // CW, CW)
    out = _irt(sid, qid, bgat, theta_w, alpha_w)
    return out.reshape(B, 1)


# speedup vs baseline: 1.0408x; 1.0408x over previous
import functools

import jax
import jax.numpy as jnp
from jax import lax
from jax.experimental import pallas as pl
from jax.experimental.pallas import tpu as pltpu
from jax.experimental.pallas import tpu_sc as plsc

NC, NS, L = 2, 16, 16
NW = NC * NS
B = 16384
D = 64
BPW = B // NW
CW = 128
NCHUNK = BPW // CW
GPC = CW // L

_LOG1P_C = (
    0.999998763883492,
    -0.4998719252774506,
    0.33112058367396247,
    -0.2351488240931608,
    0.14943483645817232,
    -0.06658820573216659,
    0.01420285926015165,
)


def _softplus(x):
    z = jnp.exp(-jnp.abs(x))
    p = jnp.float32(_LOG1P_C[-1])
    for c in _LOG1P_C[-2::-1]:
        p = p * z + jnp.float32(c)
    return jnp.maximum(x, jnp.float32(0.0)) + p * z


def _irt_body(sid_hbm, qid_hbm, theta_hbm, alpha_hbm, beta_hbm, out_hbm,
              sid_v, qid_v, theta_v, alpha_v, beta_v, out_v, sem0, sem1):
    wid = lax.axis_index("s") * NC + lax.axis_index("c")

    pltpu.sync_copy(sid_hbm.at[pl.ds(wid * NCHUNK, NCHUNK)], sid_v)
    pltpu.sync_copy(qid_hbm.at[pl.ds(wid * NCHUNK, NCHUNK)], qid_v)

    sems = (sem0, sem1)

    def fire(j):
        s = sems[j % 2]
        return (
            pltpu.async_copy(
                theta_hbm.at[sid_v.at[j]], theta_v.at[pl.ds(j * CW, CW)], s),
            pltpu.async_copy(
                alpha_hbm.at[qid_v.at[j]], alpha_v.at[pl.ds(j * CW, CW)], s),
            pltpu.async_copy(
                beta_hbm.at[qid_v.at[j]], beta_v.at[pl.ds(j * CW, CW)], s),
        )

    iota = lax.iota(jnp.int32, L)
    inflight = fire(0)
    for j in range(NCHUNK):
        nxt = fire(j + 1) if j + 1 < NCHUNK else None
        for c in inflight:
            c.wait()
        inflight = nxt

        def group(g, _):
            rows = iota + (j * CW + g * L)
            acc = jnp.zeros((L,), jnp.float32)
            for d in range(D):
                cols = jnp.full((L,), d, jnp.int32)
                th = plsc.load_gather(theta_v, [rows, cols])
                al = plsc.load_gather(alpha_v, [rows, cols])
                acc = acc + _softplus(al) * th
            logit = acc + plsc.load_gather(beta_v, [rows])
            plsc.store_scatter(out_v, [rows], logit)
            return 0

        lax.fori_loop(0, GPC, group, 0)

    pltpu.sync_copy(out_v, out_hbm.at[pl.ds(wid * BPW, BPW)])


@jax.jit
def _irt(sid, qid, theta_w, alpha_w, beta1d):
    mesh = plsc.VectorSubcoreMesh(
        core_axis_name="c", subcore_axis_name="s",
        num_cores=NC, num_subcores=NS)
    run = pl.kernel(
        _irt_body,
        out_type=jax.ShapeDtypeStruct((B,), jnp.float32),
        mesh=mesh,
        compiler_params=pltpu.CompilerParams(
            needs_layout_passes=False, use_tc_tiling_on_sc=False),
        scratch_types=[
            pltpu.VMEM((NCHUNK, CW), jnp.int32),
            pltpu.VMEM((NCHUNK, CW), jnp.int32),
            pltpu.VMEM((BPW, D), jnp.float32),
            pltpu.VMEM((BPW, D), jnp.float32),
            pltpu.VMEM((BPW,), jnp.float32),
            pltpu.VMEM((BPW,), jnp.float32),
            pltpu.SemaphoreType.DMA,
            pltpu.SemaphoreType.DMA,
        ],
    )
    logit = run(sid, qid, theta_w, alpha_w, beta1d)
    return jax.nn.sigmoid(logit.reshape(B, 1))


def kernel(student_ids, question_ids, theta_w, alpha_w, beta_w):
    sid = student_ids.astype(jnp.int32).reshape(B // CW, CW)
    qid = question_ids.astype(jnp.int32).reshape(B // CW, CW)
    return _irt(sid, qid, theta_w, alpha_w, beta_w.reshape(-1))


# speedup vs baseline: 1.1024x; 1.0592x over previous
import functools

import jax
import jax.numpy as jnp
from jax import lax
from jax.experimental import pallas as pl
from jax.experimental.pallas import tpu as pltpu
from jax.experimental.pallas import tpu_sc as plsc

NC, NS, L = 2, 16, 16
NW = NC * NS
B = 16384
D = 64
BPW = B // NW
CW = 128
NCHUNK = BPW // CW
NG = BPW // L

_LOG1P_C = (
    0.999998763883492, -0.4998719252774506, 0.33112058367396247,
    -0.2351488240931608, 0.14943483645817232, -0.06658820573216659,
    0.01420285926015165,
)


def _softplus(x):
    z = jnp.exp(-jnp.abs(x))
    p = jnp.float32(_LOG1P_C[-1])
    for c in _LOG1P_C[-2::-1]:
        p = p * z + jnp.float32(c)
    return jnp.maximum(x, jnp.float32(0.0)) + p * z


def _irt_body(sid_hbm, qid_hbm, theta_hbm, alpha_hbm, beta_hbm, out_hbm,
              sid_v, qid_v, theta_v, alpha_v, beta_v, out_v, sem0, sem1):
    wid = lax.axis_index("s") * NC + lax.axis_index("c")

    pltpu.sync_copy(sid_hbm.at[pl.ds(wid * NCHUNK, NCHUNK)], sid_v)
    pltpu.sync_copy(qid_hbm.at[pl.ds(wid * NCHUNK, NCHUNK)], qid_v)

    iota = lax.iota(jnp.int32, L)
    sems = (sem0, sem1)

    def ids_of(g):
        r = g * L + iota
        sv = plsc.load_gather(sid_v, [lax.div(r, CW), lax.rem(r, CW)])
        qv = plsc.load_gather(qid_v, [lax.div(r, CW), lax.rem(r, CW)])
        return sv, qv

    def fire(g, buf):
        sv, qv = ids_of(g)
        s = sems[buf]
        for k in range(L):
            sblk = pl.multiple_of((sv[k] // 8) * 8, 8)
            qblk = pl.multiple_of((qv[k] // 8) * 8, 8)
            pltpu.async_copy(
                theta_hbm.at[pl.ds(sblk, 8)], theta_v.at[buf, k], s)
            pltpu.async_copy(
                alpha_hbm.at[pl.ds(qblk, 8)], alpha_v.at[buf, k], s)
            pltpu.async_copy(
                beta_hbm.at[pl.ds(qblk, 8)], beta_v.at[buf, k], s)

    def drain(buf):
        for k in range(L):
            pltpu.make_async_copy(
                theta_hbm.at[pl.ds(0, 8)], theta_v.at[buf, k], sems[buf]).wait()
            pltpu.make_async_copy(
                alpha_hbm.at[pl.ds(0, 8)], alpha_v.at[buf, k], sems[buf]).wait()
            pltpu.make_async_copy(
                beta_hbm.at[pl.ds(0, 8)], beta_v.at[buf, k], sems[buf]).wait()

    def compute(g, buf):
        sv, qv = ids_of(g)
        srow = lax.rem(sv, 8)
        qrow = lax.rem(qv, 8)
        bufv = jnp.full((L,), buf, jnp.int32)
        acc = jnp.zeros((L,), jnp.float32)
        for d in range(D):
            cols = jnp.full((L,), d, jnp.int32)
            th = plsc.load_gather(theta_v, [bufv, iota, srow, cols])
            al = plsc.load_gather(alpha_v, [bufv, iota, qrow, cols])
            acc = acc + _softplus(al) * th
        bta = plsc.load_gather(beta_v, [bufv, iota, qrow])
        plsc.store_scatter(out_v, [iota + g * L], acc + bta)

    fire(0, 0)

    def pair(i, _):
        g0 = i * 2
        fire(g0 + 1, 1)
        drain(0)
        compute(g0, 0)
        fire(lax.rem(g0 + 2, NG), 0)
        drain(1)
        compute(g0 + 1, 1)
        return 0

    lax.fori_loop(0, NG // 2, pair, 0)
    drain(0)

    pltpu.sync_copy(out_v, out_hbm.at[pl.ds(wid * BPW, BPW)])


@jax.jit
def _irt(sid, qid, theta_w, alpha_w, beta_w):
    mesh = plsc.VectorSubcoreMesh(
        core_axis_name="c", subcore_axis_name="s",
        num_cores=NC, num_subcores=NS)
    run = pl.kernel(
        _irt_body,
        out_type=jax.ShapeDtypeStruct((B,), jnp.float32),
        mesh=mesh,
        compiler_params=pltpu.CompilerParams(
            needs_layout_passes=False, use_tc_tiling_on_sc=True),
        scratch_types=[
            pltpu.VMEM((NCHUNK, CW), jnp.int32),
            pltpu.VMEM((NCHUNK, CW), jnp.int32),
            pltpu.VMEM((2, L, 8, D), jnp.float32),
            pltpu.VMEM((2, L, 8, D), jnp.float32),
            pltpu.VMEM((2, L, 8), jnp.float32),
            pltpu.VMEM((BPW,), jnp.float32),
            pltpu.SemaphoreType.DMA,
            pltpu.SemaphoreType.DMA,
        ],
    )
    logit = run(sid, qid, theta_w, alpha_w, beta_w)
    return jax.nn.sigmoid(logit.reshape(B, 1))


def kernel(student_ids, question_ids, theta_w, alpha_w, beta_w):
    sid = student_ids.astype(jnp.int32).reshape(B // CW, CW)
    qid = question_ids.astype(jnp.int32).reshape(B // CW, CW)
    return _irt(sid, qid, theta_w, alpha_w, beta_w.reshape(-1))


# speedup vs baseline: 1.1914x; 1.0807x over previous
import functools

import jax
import jax.numpy as jnp
from jax import lax
from jax.experimental import pallas as pl
from jax.experimental.pallas import tpu as pltpu
from jax.experimental.pallas import tpu_sc as plsc

NC, NS, L = 2, 16, 16
NW = NC * NS
B = 16384
D = 64
BPW = B // NW
CW = 128
NCHUNK = BPW // CW
NG = BPW // L

_LOG1P_C = (
    0.999998763883492, -0.4998719252774506, 0.33112058367396247,
    -0.2351488240931608, 0.14943483645817232, -0.06658820573216659,
    0.01420285926015165,
)


def _softplus(x):
    z = jnp.exp(-jnp.abs(x))
    p = jnp.float32(_LOG1P_C[-1])
    for c in _LOG1P_C[-2::-1]:
        p = p * z + jnp.float32(c)
    return jnp.maximum(x, jnp.float32(0.0)) + p * z


def _irt_body(sid_hbm, qid_hbm, theta_hbm, alpha_hbm, beta_hbm, out_hbm,
              sid_v, qid_v, theta_v, alpha_v, beta_v, out_v, sem0, sem1):
    wid = lax.axis_index("s") * NC + lax.axis_index("c")

    pltpu.sync_copy(sid_hbm.at[pl.ds(wid * NCHUNK, NCHUNK)], sid_v)
    pltpu.sync_copy(qid_hbm.at[pl.ds(wid * NCHUNK, NCHUNK)], qid_v)

    iota = lax.iota(jnp.int32, L)
    sems = (sem0, sem1)

    def ids_of(g):
        r = g * L + iota
        sv = plsc.load_gather(sid_v, [lax.div(r, CW), lax.rem(r, CW)])
        qv = plsc.load_gather(qid_v, [lax.div(r, CW), lax.rem(r, CW)])
        return sv, qv

    def fire(g, buf):
        sv, qv = ids_of(g)
        s = sems[buf]
        for k in range(L):
            sblk = pl.multiple_of((sv[k] // 8) * 8, 8)
            qblk = pl.multiple_of((qv[k] // 8) * 8, 8)
            pltpu.async_copy(
                theta_hbm.at[pl.ds(sv[k], 1)], theta_v.at[buf, pl.ds(k, 1)], s)
            pltpu.async_copy(
                alpha_hbm.at[pl.ds(qv[k], 1)], alpha_v.at[buf, pl.ds(k, 1)], s)
            pltpu.async_copy(
                beta_hbm.at[pl.ds(qblk, 8)], beta_v.at[buf, k], s)

    def drain(buf):
        for k in range(L):
            pltpu.make_async_copy(
                theta_hbm.at[pl.ds(0, 1)], theta_v.at[buf, pl.ds(k, 1)], sems[buf]).wait()
            pltpu.make_async_copy(
                alpha_hbm.at[pl.ds(0, 1)], alpha_v.at[buf, pl.ds(k, 1)], sems[buf]).wait()
            pltpu.make_async_copy(
                beta_hbm.at[pl.ds(0, 8)], beta_v.at[buf, k], sems[buf]).wait()

    def compute(g, buf):
        sv, qv = ids_of(g)
        srow = lax.rem(sv, 8)
        qrow = lax.rem(qv, 8)
        bufv = jnp.full((L,), buf, jnp.int32)
        acc = jnp.zeros((L,), jnp.float32)
        for d in range(D):
            cols = jnp.full((L,), d, jnp.int32)
            th = plsc.load_gather(theta_v, [bufv, iota, cols])
            al = plsc.load_gather(alpha_v, [bufv, iota, cols])
            acc = acc + _softplus(al) * th
        bta = plsc.load_gather(beta_v, [bufv, iota, qrow])
        plsc.store_scatter(out_v, [iota + g * L], acc + bta)

    fire(0, 0)

    def pair(i, _):
        g0 = i * 2
        fire(g0 + 1, 1)
        drain(0)
        compute(g0, 0)
        fire(lax.rem(g0 + 2, NG), 0)
        drain(1)
        compute(g0 + 1, 1)
        return 0

    lax.fori_loop(0, NG // 2, pair, 0)
    drain(0)

    pltpu.sync_copy(out_v, out_hbm.at[pl.ds(wid * BPW, BPW)])


@jax.jit
def _irt(sid, qid, theta_w, alpha_w, beta_w):
    mesh = plsc.VectorSubcoreMesh(
        core_axis_name="c", subcore_axis_name="s",
        num_cores=NC, num_subcores=NS)
    run = pl.kernel(
        _irt_body,
        out_type=jax.ShapeDtypeStruct((B,), jnp.float32),
        mesh=mesh,
        compiler_params=pltpu.CompilerParams(
            needs_layout_passes=False, use_tc_tiling_on_sc=True),
        scratch_types=[
            pltpu.VMEM((NCHUNK, CW), jnp.int32),
            pltpu.VMEM((NCHUNK, CW), jnp.int32),
            pltpu.VMEM((2, L, D), jnp.float32),
            pltpu.VMEM((2, L, D), jnp.float32),
            pltpu.VMEM((2, L, 8), jnp.float32),
            pltpu.VMEM((BPW,), jnp.float32),
            pltpu.SemaphoreType.DMA,
            pltpu.SemaphoreType.DMA,
        ],
    )
    logit = run(sid, qid, theta_w, alpha_w, beta_w)
    return jax.nn.sigmoid(logit.reshape(B, 1))


def kernel(student_ids, question_ids, theta_w, alpha_w, beta_w):
    sid = student_ids.astype(jnp.int32).reshape(B // CW, CW)
    qid = question_ids.astype(jnp.int32).reshape(B // CW, CW)
    return _irt(sid, qid, theta_w, alpha_w, beta_w.reshape(-1))


# speedup vs baseline: 1.3170x; 1.1054x over previous
import functools

import jax
import jax.numpy as jnp
from jax import lax
from jax.experimental import pallas as pl
from jax.experimental.pallas import tpu as pltpu
from jax.experimental.pallas import tpu_sc as plsc

NC, NS, L = 2, 16, 16
NW = NC * NS
B = 16384
D = 64
BPW = B // NW
CW = 128
NCHUNK = BPW // CW
NG = BPW // L

_LOG1P_C = (
    0.999998763883492, -0.4998719252774506, 0.33112058367396247,
    -0.2351488240931608, 0.14943483645817232, -0.06658820573216659,
    0.01420285926015165,
)


def _softplus(x):
    z = jnp.exp(-jnp.abs(x))
    p = jnp.float32(_LOG1P_C[-1])
    for c in _LOG1P_C[-2::-1]:
        p = p * z + jnp.float32(c)
    return jnp.maximum(x, jnp.float32(0.0)) + p * z


def _irt_body(sid_hbm, qid_hbm, theta_hbm, alpha_hbm, beta_hbm, out_hbm,
              sid_v, qid_v, theta_v, alpha_v, beta_v, out_v, sem0, sem1):
    wid = lax.axis_index("s") * NC + lax.axis_index("c")

    pltpu.sync_copy(sid_hbm.at[pl.ds(wid * NCHUNK, NCHUNK)], sid_v)
    pltpu.sync_copy(qid_hbm.at[pl.ds(wid * NCHUNK, NCHUNK)], qid_v)

    iota = lax.iota(jnp.int32, L)
    sems = (sem0, sem1)

    def ids_of(g):
        r = g * L + iota
        sv = plsc.load_gather(sid_v, [lax.div(r, CW), lax.rem(r, CW)])
        qv = plsc.load_gather(qid_v, [lax.div(r, CW), lax.rem(r, CW)])
        return sv, qv

    def fire(g, buf):
        sv, qv = ids_of(g)
        s = sems[buf]
        for k in range(L):
            pltpu.async_copy(
                theta_hbm.at[pl.ds(sv[k], 1)], theta_v.at[buf, pl.ds(k, 1)], s)
            pltpu.async_copy(
                alpha_hbm.at[pl.ds(qv[k], 1)], alpha_v.at[buf, pl.ds(k, 1)], s)

    def drain(buf):
        for k in range(L):
            pltpu.make_async_copy(
                theta_hbm.at[pl.ds(0, 1)], theta_v.at[buf, pl.ds(k, 1)], sems[buf]).wait()
            pltpu.make_async_copy(
                alpha_hbm.at[pl.ds(0, 1)], alpha_v.at[buf, pl.ds(k, 1)], sems[buf]).wait()

    def compute(g, buf):
        sv, qv = ids_of(g)
        srow = lax.rem(sv, 8)
        qrow = lax.rem(qv, 8)
        bufv = jnp.full((L,), buf, jnp.int32)
        acc = jnp.zeros((L,), jnp.float32)
        for d in range(D):
            cols = jnp.full((L,), d, jnp.int32)
            th = plsc.load_gather(theta_v, [bufv, iota, cols])
            al = plsc.load_gather(alpha_v, [bufv, iota, cols])
            acc = acc + _softplus(al) * th
        bta = plsc.load_gather(beta_v, [iota + g * L])
        plsc.store_scatter(out_v, [iota + g * L], acc + bta)

    bcps = [pltpu.async_copy(beta_hbm.at[qid_v.at[j]],
                             beta_v.at[pl.ds(j * CW, CW)], sem0)
            for j in range(NCHUNK)]
    for c in bcps:
        c.wait()

    fire(0, 0)

    def pair(i, _):
        g0 = i * 2
        fire(g0 + 1, 1)
        drain(0)
        compute(g0, 0)
        fire(lax.rem(g0 + 2, NG), 0)
        drain(1)
        compute(g0 + 1, 1)
        return 0

    lax.fori_loop(0, NG // 2, pair, 0)
    drain(0)

    pltpu.sync_copy(out_v, out_hbm.at[pl.ds(wid * BPW, BPW)])


@jax.jit
def _irt(sid, qid, theta_w, alpha_w, beta_w):
    mesh = plsc.VectorSubcoreMesh(
        core_axis_name="c", subcore_axis_name="s",
        num_cores=NC, num_subcores=NS)
    run = pl.kernel(
        _irt_body,
        out_type=jax.ShapeDtypeStruct((B,), jnp.float32),
        mesh=mesh,
        compiler_params=pltpu.CompilerParams(
            needs_layout_passes=False, use_tc_tiling_on_sc=True),
        scratch_types=[
            pltpu.VMEM((NCHUNK, CW), jnp.int32),
            pltpu.VMEM((NCHUNK, CW), jnp.int32),
            pltpu.VMEM((2, L, D), jnp.float32),
            pltpu.VMEM((2, L, D), jnp.float32),
            pltpu.VMEM((BPW,), jnp.float32),
            pltpu.VMEM((BPW,), jnp.float32),
            pltpu.SemaphoreType.DMA,
            pltpu.SemaphoreType.DMA,
        ],
    )
    logit = run(sid, qid, theta_w, alpha_w, beta_w)
    return jax.nn.sigmoid(logit.reshape(B, 1))


def kernel(student_ids, question_ids, theta_w, alpha_w, beta_w):
    sid = student_ids.astype(jnp.int32).reshape(B // CW, CW)
    qid = question_ids.astype(jnp.int32).reshape(B // CW, CW)
    return _irt(sid, qid, theta_w, alpha_w, beta_w.reshape(-1))
